# baseline (speedup 1.0000x reference)
import functools

import jax
import jax.numpy as jnp
from jax import lax
from jax.experimental import pallas as pl
from jax.experimental.pallas import tpu as pltpu
from jax.experimental.pallas import tpu_sc as plsc

D_MODEL = 64
SCALE = 8.0

_NC = 2
_NS = 16
_NW = _NC * _NS
_LANES = 16

_B = 16384
_L = 50
_BT = _B // 128
_NUNITS = _L * _BT
_U_PER_W = _NUNITS // _NW
_DT = D_MODEL // 8

_mesh = plsc.VectorSubcoreMesh(core_axis_name="c", subcore_axis_name="s")

_TCW = 8192
_TCGRID = 123


def _tc_body(x_ref, o_ref):
    r = lax.broadcasted_iota(jnp.int32, (D_MODEL, D_MODEL), 0)
    c = lax.broadcasted_iota(jnp.int32, (D_MODEL, D_MODEL), 1)
    eye = (r == c).astype(jnp.float32)
    y = lax.dot_general(x_ref[...], eye, (((0,), (0,)), ((), ())),
                        preferred_element_type=jnp.float32) * SCALE
    y3 = y.reshape(_TCW // 2, 2, D_MODEL)
    o_ref[...] = jnp.concatenate([y3[:, 0, :], y3[:, 1, :]], axis=1)


_tc_relayout = pl.pallas_call(
    _tc_body,
    grid=(_TCGRID,),
    in_specs=[pl.BlockSpec((D_MODEL, _TCW), lambda i: (0, i))],
    out_specs=pl.BlockSpec((_TCW // 2, 128), lambda i: (i, 0)),
    out_shape=jax.ShapeDtypeStruct((500000, 128), jnp.float32),
    compiler_params=pltpu.CompilerParams(vmem_limit_bytes=110 * 2**20),
)


@functools.partial(
    pl.kernel,
    out_type=jax.ShapeDtypeStruct((_L, _DT, _BT, 1024), jnp.float32),
    mesh=_mesh,
    scratch_types=[
        pltpu.VMEM((128,), jnp.int32),
        pltpu.VMEM((128,), jnp.int32),
        pltpu.VMEM((128, D_MODEL), jnp.float32),
        pltpu.VMEM((128, D_MODEL), jnp.float32),
        pltpu.VMEM((D_MODEL * 128,), jnp.float32),
        pltpu.VMEM((D_MODEL * 128,), jnp.float32),
        pltpu.SemaphoreType.DMA,
        pltpu.SemaphoreType.DMA,
        pltpu.SemaphoreType.DMA,
        pltpu.SemaphoreType.DMA,
    ],
    compiler_params=pltpu.CompilerParams(use_tc_tiling_on_sc=False,
                                         needs_layout_passes=False),
)
def _embed(xt_hbm, table_hbm, out_hbm, i0, i1, r0, r1, s0, s1,
           g0, g1, o0, o1):
    wid = lax.axis_index("s") * _NC + lax.axis_index("c")
    ubase = wid * _U_PER_W
    idx_v = (i0, i1)
    rows_v = (r0, r1)
    stage_v = (s0, s1)
    gsem = (g0, g1)
    osem = (o0, o1)

    def fetch_start(u, b):
        l = u // _BT
        bt = u % _BT
        pltpu.sync_copy(xt_hbm.at[l, pl.ds(bt * 128, 128)], idx_v[b])
        pltpu.async_copy(table_hbm.at[idx_v[b]], rows_v[b], gsem[b])

    def fetch_wait(b):
        pltpu.make_async_copy(table_hbm.at[idx_v[b]], rows_v[b],
                              gsem[b]).wait()

    def out_start(u, b):
        l = u // _BT
        bt = u % _BT
        for dt in range(_DT):
            pltpu.async_copy(stage_v[b].at[pl.ds(dt * 1024, 1024)],
                             out_hbm.at[l, dt, bt], osem[b])

    def out_wait(b):
        for dt in range(_DT):
            pltpu.make_async_copy(stage_v[b].at[pl.ds(dt * 1024, 1024)],
                                  out_hbm.at[0, dt, 0],
                                  osem[b]).wait()

    iot = lax.iota(jnp.int32, _LANES)
    pats = [(iot + k) & 15 for k in range(16)]
    spats = [((iot + k) & 15) * 128 + iot for k in range(16)]

    def transpose_scale(b):
        rows = rows_v[b]
        stage = stage_v[b]

        @plsc.parallel_loop(0, 32, unroll=2)
        def _blk(blk):
            bl0 = (blk & 7) * 16
            c0 = lax.shift_right_logical(blk, 3) * 16
            row_ids = bl0 + iot
            sbase = c0 * 128 + bl0
            for k in range(16):
                cols = c0 + pats[k]
                v = plsc.load_gather(rows, [row_ids, cols])
                plsc.store_scatter(stage, [sbase + spats[k]], v)

    fetch_start(ubase, 0)
    fetch_start(ubase + 1, 1)
    fetch_wait(0)
    transpose_scale(0)
    out_start(ubase, 0)

    def pair_body(k, carry):
        for (g_off, b) in ((1, 1), (2, 0)):
            u = ubase + 2 * k + g_off
            nb = 1 - b
            out_wait(nb)
            fetch_start(u + 1, nb)
            fetch_wait(b)
            transpose_scale(b)
            out_start(u, b)
        return carry

    lax.fori_loop(0, (_U_PER_W - 2) // 2, pair_body, 0)

    fetch_wait(1)
    transpose_scale(1)
    out_start(ubase + _U_PER_W - 1, 1)
    out_wait(0)
    out_wait(1)


def kernel(x, table):
    xt = x.T
    t_lin = _tc_relayout(table.T).reshape(1000000, 64)
    raw = _embed(xt, t_lin)
    raw5 = raw.reshape(_L, _DT, _BT, 8, 128)
    return raw5.transpose(2, 4, 0, 1, 3).reshape(_B, _L, D_MODEL)


# speedup vs baseline: 1.0424x; 1.0424x over previous
import functools

import jax
import jax.numpy as jnp
from jax import lax
from jax.experimental import pallas as pl
from jax.experimental.pallas import tpu as pltpu
from jax.experimental.pallas import tpu_sc as plsc

D_MODEL = 64
SCALE = 8.0

_NC = 2
_NS = 16
_NW = _NC * _NS
_LANES = 16

_B = 16384
_L = 50
_BT = _B // 128
_NUNITS = _L * _BT
_U_PER_W = _NUNITS // _NW
_DT = D_MODEL // 8

_mesh = plsc.VectorSubcoreMesh(core_axis_name="c", subcore_axis_name="s")

_TCW = 8192
_TCGRID = 123


def _tc_body(x_ref, o_ref):
    y3 = (x_ref[...].T * SCALE).reshape(_TCW // 2, 2, D_MODEL)
    o_ref[...] = jnp.concatenate([y3[:, 0, :], y3[:, 1, :]], axis=1)


_tc_relayout = pl.pallas_call(
    _tc_body,
    grid=(_TCGRID,),
    in_specs=[pl.BlockSpec((D_MODEL, _TCW), lambda i: (0, i))],
    out_specs=pl.BlockSpec((_TCW // 2, 128), lambda i: (i, 0)),
    out_shape=jax.ShapeDtypeStruct((500000, 128), jnp.float32),
    compiler_params=pltpu.CompilerParams(vmem_limit_bytes=110 * 2**20),
)


@functools.partial(
    pl.kernel,
    out_type=jax.ShapeDtypeStruct((_L, _DT, _BT, 1024), jnp.float32),
    mesh=_mesh,
    scratch_types=[
        pltpu.VMEM((128,), jnp.int32),
        pltpu.VMEM((128,), jnp.int32),
        pltpu.VMEM((128, D_MODEL), jnp.float32),
        pltpu.VMEM((128, D_MODEL), jnp.float32),
        pltpu.VMEM((D_MODEL * 128,), jnp.float32),
        pltpu.VMEM((D_MODEL * 128,), jnp.float32),
        pltpu.SemaphoreType.DMA,
        pltpu.SemaphoreType.DMA,
        pltpu.SemaphoreType.DMA,
        pltpu.SemaphoreType.DMA,
    ],
    compiler_params=pltpu.CompilerParams(use_tc_tiling_on_sc=False,
                                         needs_layout_passes=False),
)
def _embed(xt_hbm, table_hbm, out_hbm, i0, i1, r0, r1, s0, s1,
           g0, g1, o0, o1):
    wid = lax.axis_index("s") * _NC + lax.axis_index("c")
    ubase = wid * _U_PER_W
    idx_v = (i0, i1)
    rows_v = (r0, r1)
    stage_v = (s0, s1)
    gsem = (g0, g1)
    osem = (o0, o1)

    def fetch_start(u, b):
        l = u // _BT
        bt = u % _BT
        pltpu.sync_copy(xt_hbm.at[l, pl.ds(bt * 128, 128)], idx_v[b])
        pltpu.async_copy(table_hbm.at[idx_v[b]], rows_v[b], gsem[b])

    def fetch_wait(b):
        pltpu.make_async_copy(table_hbm.at[idx_v[b]], rows_v[b],
                              gsem[b]).wait()

    def out_start(u, b):
        l = u // _BT
        bt = u % _BT
        for dt in range(_DT):
            pltpu.async_copy(stage_v[b].at[pl.ds(dt * 1024, 1024)],
                             out_hbm.at[l, dt, bt], osem[b])

    def out_wait(b):
        for dt in range(_DT):
            pltpu.make_async_copy(stage_v[b].at[pl.ds(dt * 1024, 1024)],
                                  out_hbm.at[0, dt, 0],
                                  osem[b]).wait()

    iot = lax.iota(jnp.int32, _LANES)
    pats = [(iot + k) & 15 for k in range(16)]
    spats = [((iot + k) & 15) * 128 + iot for k in range(16)]

    def transpose_scale(b):
        rows = rows_v[b]
        stage = stage_v[b]

        @plsc.parallel_loop(0, 32, unroll=2)
        def _blk(blk):
            bl0 = (blk & 7) * 16
            c0 = lax.shift_right_logical(blk, 3) * 16
            row_ids = bl0 + iot
            sbase = c0 * 128 + bl0
            for k in range(16):
                cols = c0 + pats[k]
                v = plsc.load_gather(rows, [row_ids, cols])
                plsc.store_scatter(stage, [sbase + spats[k]], v)

    fetch_start(ubase, 0)
    fetch_start(ubase + 1, 1)
    fetch_wait(0)
    transpose_scale(0)
    out_start(ubase, 0)

    def pair_body(k, carry):
        for (g_off, b) in ((1, 1), (2, 0)):
            u = ubase + 2 * k + g_off
            nb = 1 - b
            out_wait(nb)
            fetch_start(u + 1, nb)
            fetch_wait(b)
            transpose_scale(b)
            out_start(u, b)
        return carry

    lax.fori_loop(0, (_U_PER_W - 2) // 2, pair_body, 0)

    fetch_wait(1)
    transpose_scale(1)
    out_start(ubase + _U_PER_W - 1, 1)
    out_wait(0)
    out_wait(1)


def kernel(x, table):
    xt = x.T
    t_lin = _tc_relayout(table.T).reshape(1000000, 64)
    raw = _embed(xt, t_lin)
    raw5 = raw.reshape(_L, _DT, _BT, 8, 128)
    return raw5.transpose(2, 4, 0, 1, 3).reshape(_B, _L, D_MODEL)


# speedup vs baseline: 1.2014x; 1.1525x over previous
import functools

import jax
import jax.numpy as jnp
from jax import lax
from jax.experimental import pallas as pl
from jax.experimental.pallas import tpu as pltpu
from jax.experimental.pallas import tpu_sc as plsc

D_MODEL = 64
SCALE = 8.0

_NC = 2
_NS = 16
_NW = _NC * _NS
_LANES = 16

_B = 16384
_L = 50
_BT = _B // 128
_NUNITS = _L * _BT
_U_PER_W = _NUNITS // _NW
_DT = D_MODEL // 8

_mesh = plsc.VectorSubcoreMesh(core_axis_name="c", subcore_axis_name="s")

_TCW = 8192
_TCGRID = 123


def _tc_body(x_ref, o_ref):
    y3 = (x_ref[...].T * SCALE).reshape(_TCW // 2, 2, D_MODEL)
    o_ref[...] = jnp.concatenate([y3[:, 0, :], y3[:, 1, :]], axis=1)


_tc_relayout = pl.pallas_call(
    _tc_body,
    grid=(_TCGRID,),
    in_specs=[pl.BlockSpec((D_MODEL, _TCW), lambda i: (0, i))],
    out_specs=pl.BlockSpec((_TCW // 2, 128), lambda i: (i, 0)),
    out_shape=jax.ShapeDtypeStruct((500000, 128), jnp.float32),
    compiler_params=pltpu.CompilerParams(vmem_limit_bytes=110 * 2**20),
)


@functools.partial(
    pl.kernel,
    out_type=jax.ShapeDtypeStruct((_L, _DT, _BT, 1024), jnp.float32),
    mesh=_mesh,
    scratch_types=[
        pltpu.VMEM((_U_PER_W * 128,), jnp.int32),
        pltpu.VMEM((128, D_MODEL), jnp.float32),
        pltpu.VMEM((128, D_MODEL), jnp.float32),
        pltpu.VMEM((D_MODEL * 128,), jnp.float32),
        pltpu.VMEM((D_MODEL * 128,), jnp.float32),
        pltpu.SemaphoreType.DMA,
        pltpu.SemaphoreType.DMA,
        pltpu.SemaphoreType.DMA,
        pltpu.SemaphoreType.DMA,
    ],
    compiler_params=pltpu.CompilerParams(use_tc_tiling_on_sc=False,
                                         needs_layout_passes=False),
)
def _embed(xt_hbm, table_hbm, out_hbm, idx_all, r0, r1, s0, s1,
           g0, g1, o0, o1):
    wid = lax.axis_index("s") * _NC + lax.axis_index("c")
    ubase = wid * _U_PER_W
    rows_v = (r0, r1)
    stage_v = (s0, s1)
    gsem = (g0, g1)
    osem = (o0, o1)

    def fetch_start(j, b):
        idx = idx_all.at[pl.ds(j * 128, 128)]
        pltpu.async_copy(table_hbm.at[idx], rows_v[b], gsem[b])

    def fetch_wait(b):
        idx = idx_all.at[pl.ds(0, 128)]
        pltpu.make_async_copy(table_hbm.at[idx], rows_v[b],
                              gsem[b]).wait()

    def out_start(u, b):
        l = u // _BT
        bt = u % _BT
        for dt in range(_DT):
            pltpu.async_copy(stage_v[b].at[pl.ds(dt * 1024, 1024)],
                             out_hbm.at[l, dt, bt], osem[b])

    def out_wait(b):
        for dt in range(_DT):
            pltpu.make_async_copy(stage_v[b].at[pl.ds(dt * 1024, 1024)],
                                  out_hbm.at[0, dt, 0],
                                  osem[b]).wait()

    iot = lax.iota(jnp.int32, _LANES)
    pats = [(iot + k) & 15 for k in range(16)]
    spats = [((iot + k) & 15) * 128 + iot for k in range(16)]

    def transpose_scale(b):
        rows = rows_v[b]
        stage = stage_v[b]

        @plsc.parallel_loop(0, 32, unroll=2)
        def _blk(blk):
            bl0 = (blk & 7) * 16
            c0 = lax.shift_right_logical(blk, 3) * 16
            row_ids = bl0 + iot
            sbase = c0 * 128 + bl0
            for k in range(16):
                cols = c0 + pats[k]
                v = plsc.load_gather(rows, [row_ids, cols])
                plsc.store_scatter(stage, [sbase + spats[k]], v)

    pltpu.sync_copy(xt_hbm.at[pl.ds(ubase * 128, _U_PER_W * 128)], idx_all)

    fetch_start(0, 0)
    fetch_start(1, 1)
    fetch_wait(0)
    transpose_scale(0)
    out_start(ubase, 0)

    def pair_body(k, carry):
        for (g_off, b) in ((1, 1), (2, 0)):
            j = 2 * k + g_off
            u = ubase + j
            nb = 1 - b
            out_wait(nb)
            fetch_start(j + 1, nb)
            fetch_wait(b)
            transpose_scale(b)
            out_start(u, b)
        return carry

    lax.fori_loop(0, (_U_PER_W - 2) // 2, pair_body, 0)

    fetch_wait(1)
    transpose_scale(1)
    out_start(ubase + _U_PER_W - 1, 1)
    out_wait(0)
    out_wait(1)


def kernel(x, table):
    xt = x.T.reshape(-1)
    t_lin = _tc_relayout(table.T).reshape(1000000, 64)
    raw = _embed(xt, t_lin)
    raw5 = raw.reshape(_L, _DT, _BT, 8, 128)
    return raw5.transpose(2, 4, 0, 1, 3).reshape(_B, _L, D_MODEL)


# speedup vs baseline: 1.2079x; 1.0054x over previous
import functools

import jax
import jax.numpy as jnp
from jax import lax
from jax.experimental import pallas as pl
from jax.experimental.pallas import tpu as pltpu
from jax.experimental.pallas import tpu_sc as plsc

D_MODEL = 64
SCALE = 8.0

_NC = 2
_NS = 16
_NW = _NC * _NS
_LANES = 16

_B = 16384
_L = 50
_BT = _B // 128
_NUNITS = _L * _BT
_U_PER_W = _NUNITS // _NW
_DT = D_MODEL // 8

_mesh = plsc.VectorSubcoreMesh(core_axis_name="c", subcore_axis_name="s")

_TCW = 16384
_TCGRID = 62


def _tc_body(x_ref, o_ref):
    y3 = (x_ref[...].T * SCALE).reshape(_TCW // 2, 2, D_MODEL)
    o_ref[...] = jnp.concatenate([y3[:, 0, :], y3[:, 1, :]], axis=1)


_tc_relayout = pl.pallas_call(
    _tc_body,
    grid=(_TCGRID,),
    in_specs=[pl.BlockSpec((D_MODEL, _TCW), lambda i: (0, i))],
    out_specs=pl.BlockSpec((_TCW // 2, 128), lambda i: (i, 0)),
    out_shape=jax.ShapeDtypeStruct((500000, 128), jnp.float32),
    compiler_params=pltpu.CompilerParams(vmem_limit_bytes=110 * 2**20),
)


@functools.partial(
    pl.kernel,
    out_type=jax.ShapeDtypeStruct((_L, _DT, _BT, 1024), jnp.float32),
    mesh=_mesh,
    scratch_types=[
        pltpu.VMEM((_U_PER_W * 128,), jnp.int32),
        pltpu.VMEM((128, D_MODEL), jnp.float32),
        pltpu.VMEM((128, D_MODEL), jnp.float32),
        pltpu.VMEM((D_MODEL * 128,), jnp.float32),
        pltpu.VMEM((D_MODEL * 128,), jnp.float32),
        pltpu.SemaphoreType.DMA,
        pltpu.SemaphoreType.DMA,
        pltpu.SemaphoreType.DMA,
        pltpu.SemaphoreType.DMA,
    ],
    compiler_params=pltpu.CompilerParams(use_tc_tiling_on_sc=False,
                                         needs_layout_passes=False),
)
def _embed(xt_hbm, table_hbm, out_hbm, idx_all, r0, r1, s0, s1,
           g0, g1, o0, o1):
    wid = lax.axis_index("s") * _NC + lax.axis_index("c")
    ubase = wid * _U_PER_W
    rows_v = (r0, r1)
    stage_v = (s0, s1)
    gsem = (g0, g1)
    osem = (o0, o1)

    def fetch_start(j, b):
        idx = idx_all.at[pl.ds(j * 128, 128)]
        pltpu.async_copy(table_hbm.at[idx], rows_v[b], gsem[b])

    def fetch_wait(b):
        idx = idx_all.at[pl.ds(0, 128)]
        pltpu.make_async_copy(table_hbm.at[idx], rows_v[b],
                              gsem[b]).wait()

    def out_start(u, b):
        l = u // _BT
        bt = u % _BT
        for dt in range(_DT):
            pltpu.async_copy(stage_v[b].at[pl.ds(dt * 1024, 1024)],
                             out_hbm.at[l, dt, bt], osem[b])

    def out_wait(b):
        for dt in range(_DT):
            pltpu.make_async_copy(stage_v[b].at[pl.ds(dt * 1024, 1024)],
                                  out_hbm.at[0, dt, 0],
                                  osem[b]).wait()

    iot = lax.iota(jnp.int32, _LANES)
    pats = [(iot + k) & 15 for k in range(16)]
    spats = [((iot + k) & 15) * 128 + iot for k in range(16)]

    def transpose_scale(b):
        rows = rows_v[b]
        stage = stage_v[b]

        @plsc.parallel_loop(0, 32, unroll=2)
        def _blk(blk):
            bl0 = (blk & 7) * 16
            c0 = lax.shift_right_logical(blk, 3) * 16
            row_ids = bl0 + iot
            sbase = c0 * 128 + bl0
            for k in range(16):
                cols = c0 + pats[k]
                v = plsc.load_gather(rows, [row_ids, cols])
                plsc.store_scatter(stage, [sbase + spats[k]], v)

    pltpu.sync_copy(xt_hbm.at[pl.ds(ubase * 128, _U_PER_W * 128)], idx_all)

    fetch_start(0, 0)
    fetch_start(1, 1)
    fetch_wait(0)
    transpose_scale(0)
    out_start(ubase, 0)

    def pair_body(k, carry):
        for (g_off, b) in ((1, 1), (2, 0)):
            j = 2 * k + g_off
            u = ubase + j
            nb = 1 - b
            out_wait(nb)
            fetch_start(j + 1, nb)
            fetch_wait(b)
            transpose_scale(b)
            out_start(u, b)
        return carry

    lax.fori_loop(0, (_U_PER_W - 2) // 2, pair_body, 0)

    fetch_wait(1)
    transpose_scale(1)
    out_start(ubase + _U_PER_W - 1, 1)
    out_wait(0)
    out_wait(1)


def kernel(x, table):
    xt = x.T.reshape(-1)
    t_lin = _tc_relayout(table.T).reshape(1000000, 64)
    raw = _embed(xt, t_lin)
    raw5 = raw.reshape(_L, _DT, _BT, 8, 128)
    return raw5.transpose(2, 4, 0, 1, 3).reshape(_B, _L, D_MODEL)


# speedup vs baseline: 1.2843x; 1.0632x over previous
import functools

import jax
import jax.numpy as jnp
from jax import lax
from jax.experimental import pallas as pl
from jax.experimental.pallas import tpu as pltpu
from jax.experimental.pallas import tpu_sc as plsc

D_MODEL = 64
SCALE = 8.0

_NC = 2
_NS = 16
_NW = _NC * _NS
_LANES = 16

_B = 16384
_L = 50
_BT = _B // 128
_NUNITS = _L * _BT
_U_PER_W = _NUNITS // _NW
_DT = D_MODEL // 8

_mesh = plsc.VectorSubcoreMesh(core_axis_name="c", subcore_axis_name="s")

_TCW = 16384
_TCGRID = 62


def _tc_body(x_ref, o_ref):
    y3 = (x_ref[...].T * SCALE).reshape(_TCW // 2, 2, D_MODEL)
    o_ref[...] = jnp.concatenate([y3[:, 0, :], y3[:, 1, :]], axis=1)


_tc_relayout = pl.pallas_call(
    _tc_body,
    grid=(_TCGRID,),
    in_specs=[pl.BlockSpec((D_MODEL, _TCW), lambda i: (0, i))],
    out_specs=pl.BlockSpec((_TCW // 2, 128), lambda i: (i, 0)),
    out_shape=jax.ShapeDtypeStruct((500000, 128), jnp.float32),
    compiler_params=pltpu.CompilerParams(vmem_limit_bytes=110 * 2**20),
)


@functools.partial(
    pl.kernel,
    out_type=jax.ShapeDtypeStruct((_L, _DT, _BT, 1024), jnp.float32),
    mesh=_mesh,
    scratch_types=[
        pltpu.VMEM((_U_PER_W * 128,), jnp.int32),
        pltpu.VMEM((128, D_MODEL), jnp.float32),
        pltpu.VMEM((128, D_MODEL), jnp.float32),
        pltpu.VMEM((D_MODEL * 128,), jnp.float32),
        pltpu.VMEM((D_MODEL * 128,), jnp.float32),
        pltpu.SemaphoreType.DMA,
        pltpu.SemaphoreType.DMA,
        pltpu.SemaphoreType.DMA,
        pltpu.SemaphoreType.DMA,
    ],
    compiler_params=pltpu.CompilerParams(use_tc_tiling_on_sc=False,
                                         needs_layout_passes=False),
)
def _embed(xt_hbm, table_hbm, out_hbm, idx_all, r0, r1, s0, s1,
           g0, g1, o0, o1):
    wid = lax.axis_index("s") * _NC + lax.axis_index("c")
    ubase = wid * _U_PER_W
    rows_v = (r0, r1)
    stage_v = (s0, s1)
    gsem = (g0, g1)
    osem = (o0, o1)

    def fetch_start(j, b):
        idx = idx_all.at[pl.ds(j * 128, 128)]
        pltpu.async_copy(table_hbm.at[idx], rows_v[b], gsem[b])

    def fetch_wait(b):
        idx = idx_all.at[pl.ds(0, 128)]
        pltpu.make_async_copy(table_hbm.at[idx], rows_v[b],
                              gsem[b]).wait()

    def out_start(u, b):
        l = u // _BT
        bt = u % _BT
        for dt in range(_DT):
            pltpu.async_copy(stage_v[b].at[pl.ds(dt * 1024, 1024)],
                             out_hbm.at[l, dt, bt], osem[b])

    def out_wait(b):
        for dt in range(_DT):
            pltpu.make_async_copy(stage_v[b].at[pl.ds(dt * 1024, 1024)],
                                  out_hbm.at[0, dt, 0],
                                  osem[b]).wait()

    iot = lax.iota(jnp.int32, _LANES)
    zvec = iot & 0
    gpats = [iot * D_MODEL + ((iot + k) & 15) for k in range(16)]
    spats = [((iot + k) & 15) * 128 + iot for k in range(16)]

    def transpose_scale(b):
        rows = rows_v[b]
        stage = stage_v[b]

        @plsc.parallel_loop(0, 32, unroll=2)
        def _blk(blk):
            bl0 = (blk & 7) * 16
            c0 = lax.shift_right_logical(blk, 3) * 16
            gbase = bl0 * D_MODEL + c0
            sbase = c0 * 128 + bl0
            for k in range(16):
                v = plsc.load_gather(rows, [zvec, gbase + gpats[k]])
                plsc.store_scatter(stage, [sbase + spats[k]], v)

    pltpu.sync_copy(xt_hbm.at[pl.ds(ubase * 128, _U_PER_W * 128)], idx_all)

    fetch_start(0, 0)
    fetch_start(1, 1)
    fetch_wait(0)
    transpose_scale(0)
    out_start(ubase, 0)

    def pair_body(k, carry):
        for (g_off, b) in ((1, 1), (2, 0)):
            j = 2 * k + g_off
            u = ubase + j
            nb = 1 - b
            out_wait(nb)
            fetch_start(j + 1, nb)
            fetch_wait(b)
            transpose_scale(b)
            out_start(u, b)
        return carry

    lax.fori_loop(0, (_U_PER_W - 2) // 2, pair_body, 0)

    fetch_wait(1)
    transpose_scale(1)
    out_start(ubase + _U_PER_W - 1, 1)
    out_wait(0)
    out_wait(1)


def kernel(x, table):
    xt = x.T.reshape(-1)
    t_lin = _tc_relayout(table.T).reshape(1000000, 64)
    raw = _embed(xt, t_lin)
    raw5 = raw.reshape(_L, _DT, _BT, 8, 128)
    return raw5.transpose(2, 4, 0, 1, 3).reshape(_B, _L, D_MODEL)
